# baseline (speedup 1.0000x reference)
import functools

import jax
import jax.numpy as jnp
from jax import lax
from jax.experimental import pallas as pl
from jax.experimental.pallas import tpu as pltpu
from jax.experimental.pallas import tpu_sc as plsc

N = 32768
D = 512
B = 16
NC = 2
NS = 16
NW = NC * NS
L = 16
KV = D // L
RPT = N // NW
CH = 64
NCH = RPT // CH

_mesh = plsc.VectorSubcoreMesh(core_axis_name="c", subcore_axis_name="s")


@functools.partial(
    pl.kernel,
    mesh=_mesh,
    out_type=jax.ShapeDtypeStruct((NC, B, D), jnp.float32),
    scratch_types=[
        pltpu.VMEM((CH, D), jnp.float32),
        pltpu.VMEM((CH, D), jnp.float32),
        pltpu.VMEM((B, D), jnp.float32),
        pltpu.VMEM((NS, D), jnp.float32),
        pltpu.VMEM((32,), jnp.int32),
        pltpu.VMEM_SHARED((B, NS, D), jnp.float32),
        pltpu.SemaphoreType.DMA,
        pltpu.SemaphoreType.DMA,
    ],
)
def _jagged_max(values_hbm, ps_hbm, out_hbm,
                buf0, buf1, partial, mbuf, ps_v, shared, sem0, sem1):
    cid = lax.axis_index("c")
    sid = lax.axis_index("s")
    wid = sid * NC + cid
    row0 = wid * RPT

    pltpu.sync_copy(ps_hbm, ps_v)
    pvec0 = ps_v[pl.ds(0, L)]
    pvec1 = ps_v[pl.ds(L, L)]
    ps_s = [pvec0[i] for i in range(L)] + [pvec1[0]]

    neg = jnp.full((L,), -jnp.inf, jnp.float32)
    for s in range(B):
        for k in range(KV):
            partial[s, pl.ds(k * L, L)] = neg

    def start(j, buf, sem):
        pltpu.async_copy(values_hbm.at[pl.ds(row0 + j * CH, CH), :], buf, sem)

    def wait(buf, sem):
        pltpu.make_async_copy(
            values_hbm.at[pl.ds(row0, CH), :], buf, sem).wait()

    start(0, buf0, sem0)
    start(1, buf1, sem1)

    def process(j, buf):
        chunk_lo = row0 + j * CH
        for s in range(B):
            a = jnp.maximum(ps_s[s], chunk_lo) - chunk_lo
            b = jnp.minimum(ps_s[s + 1], chunk_lo + CH) - chunk_lo

            @pl.when(b > a)
            def _():
                for half in range(2):
                    k0 = half * (KV // 2)
                    acc0 = tuple(partial[s, pl.ds((k0 + k) * L, L)]
                                 for k in range(KV // 2))

                    def rbody(r, acc):
                        return tuple(
                            jnp.maximum(acc[k], buf[r, pl.ds((k0 + k) * L, L)])
                            for k in range(KV // 2))

                    acc = lax.fori_loop(a, b, rbody, acc0)
                    for k in range(KV // 2):
                        partial[s, pl.ds((k0 + k) * L, L)] = acc[k]

    def loop_body(jj, carry):
        j = 2 * jj
        wait(buf0, sem0)
        process(j, buf0)

        @pl.when(j + 2 < NCH)
        def _():
            start(j + 2, buf0, sem0)

        wait(buf1, sem1)
        process(j + 1, buf1)

        @pl.when(j + 3 < NCH)
        def _():
            start(j + 3, buf1, sem1)

        return carry

    lax.fori_loop(0, NCH // 2, loop_body, 0)

    for s in range(B):
        pltpu.sync_copy(partial.at[s], shared.at[s, sid])
    plsc.subcore_barrier()
    pltpu.sync_copy(shared.at[sid], mbuf)
    for k in range(KV):
        acc = mbuf[0, pl.ds(k * L, L)]
        for t in range(1, NS):
            acc = jnp.maximum(acc, mbuf[t, pl.ds(k * L, L)])
        partial[0, pl.ds(k * L, L)] = acc
    pltpu.sync_copy(partial.at[0], out_hbm.at[cid, sid])


@jax.jit
def kernel(values, prefix_sum):
    ps = jnp.pad(prefix_sum, (0, 32 - (B + 1)), mode="edge")
    halves = _jagged_max(values, ps)
    return jnp.maximum(halves[0], halves[1])


# speedup vs baseline: 1.3354x; 1.3354x over previous
import functools

import jax
import jax.numpy as jnp
from jax import lax
from jax.experimental import pallas as pl
from jax.experimental.pallas import tpu as pltpu
from jax.experimental.pallas import tpu_sc as plsc

N = 32768
D = 512
B = 16
S_TC = 24576
SC_ROWS = N - S_TC
NC = 2
NS = 16
L = 16
CPC = D // NC
KV = CPC // L
RPT = SC_ROWS // NS
CH = 128
NCH = RPT // CH
RB = 512

_mesh = plsc.VectorSubcoreMesh(core_axis_name="c", subcore_axis_name="s")


@functools.partial(
    pl.kernel,
    mesh=_mesh,
    out_type=jax.ShapeDtypeStruct((B, D), jnp.float32),
    scratch_types=[
        pltpu.VMEM((CH, CPC), jnp.float32),
        pltpu.VMEM((CH, CPC), jnp.float32),
        pltpu.VMEM((B, CPC), jnp.float32),
        pltpu.VMEM((NS, CPC), jnp.float32),
        pltpu.VMEM((32,), jnp.int32),
        pltpu.VMEM_SHARED((B, NS, CPC), jnp.float32),
        pltpu.SemaphoreType.DMA,
        pltpu.SemaphoreType.DMA,
    ],
)
def _jagged_max_sc(values_hbm, ps_hbm, out_hbm,
                   buf0, buf1, partial, mbuf, ps_v, shared, sem0, sem1):
    cid = lax.axis_index("c")
    sid = lax.axis_index("s")
    c0 = cid * CPC
    row0 = S_TC + sid * RPT

    pltpu.sync_copy(ps_hbm, ps_v)
    pvec0 = ps_v[pl.ds(0, L)]
    pvec1 = ps_v[pl.ds(L, L)]
    ps_s = [pvec0[i] for i in range(L)] + [pvec1[0]]

    neg = jnp.full((L,), -jnp.inf, jnp.float32)
    for s in range(B):
        for k in range(KV):
            partial[s, pl.ds(k * L, L)] = neg

    def start(j, buf, sem):
        pltpu.async_copy(
            values_hbm.at[pl.ds(row0 + j * CH, CH), pl.ds(c0, CPC)], buf, sem)

    def wait(buf, sem):
        pltpu.make_async_copy(
            values_hbm.at[pl.ds(row0, CH), pl.ds(c0, CPC)], buf, sem).wait()

    start(0, buf0, sem0)
    start(1, buf1, sem1)

    def process(j, buf):
        chunk_lo = row0 + j * CH
        for s in range(B):
            a = jnp.maximum(ps_s[s], chunk_lo) - chunk_lo
            b = jnp.minimum(ps_s[s + 1], chunk_lo + CH) - chunk_lo

            @pl.when(b > a)
            def _():
                acc0 = tuple(partial[s, pl.ds(k * L, L)] for k in range(KV))

                def rbody(r, acc):
                    return tuple(
                        jnp.maximum(acc[k], buf[r, pl.ds(k * L, L)])
                        for k in range(KV))

                acc = lax.fori_loop(a, b, rbody, acc0)
                for k in range(KV):
                    partial[s, pl.ds(k * L, L)] = acc[k]

    def loop_body(jj, carry):
        j = 2 * jj
        wait(buf0, sem0)
        process(j, buf0)

        @pl.when(j + 2 < NCH)
        def _():
            start(j + 2, buf0, sem0)

        wait(buf1, sem1)
        process(j + 1, buf1)

        @pl.when(j + 3 < NCH)
        def _():
            start(j + 3, buf1, sem1)

        return carry

    lax.fori_loop(0, NCH // 2, loop_body, 0)

    for s in range(B):
        pltpu.sync_copy(partial.at[s], shared.at[s, sid])
    plsc.subcore_barrier()
    pltpu.sync_copy(shared.at[sid], mbuf)
    for k in range(KV):
        acc = mbuf[0, pl.ds(k * L, L)]
        for t in range(1, NS):
            acc = jnp.maximum(acc, mbuf[t, pl.ds(k * L, L)])
        partial[0, pl.ds(k * L, L)] = acc
    pltpu.sync_copy(partial.at[0], out_hbm.at[sid, pl.ds(c0, CPC)])


def _tc_body(ps_ref, x_ref, o_ref, acc_ref):
    i = pl.program_id(0)
    nsteps = pl.num_programs(0)

    @pl.when(i == 0)
    def _():
        acc_ref[...] = jnp.full_like(acc_ref, -jnp.inf)

    base = i * RB
    iota = lax.broadcasted_iota(jnp.int32, (RB, D), 0)
    x = x_ref[...]
    for s in range(B):
        a = jnp.maximum(ps_ref[s] - base, 0)
        b = jnp.minimum(ps_ref[s + 1] - base, RB)

        @pl.when(b > a)
        def _():
            mask = (iota >= a) & (iota < b)
            masked = jnp.where(mask, x, -jnp.inf)
            m8 = jnp.max(masked.reshape(RB // 8, 8, D), axis=0)
            acc_ref[s] = jnp.maximum(acc_ref[s], m8)

    @pl.when(i == nsteps - 1)
    def _():
        o_ref[...] = jnp.max(acc_ref[...], axis=1)


_tc_prefix = pl.pallas_call(
    _tc_body,
    grid=(S_TC // RB,),
    in_specs=[pl.BlockSpec(memory_space=pltpu.SMEM),
              pl.BlockSpec((RB, D), lambda i: (i, 0))],
    out_specs=pl.BlockSpec((B, D), lambda i: (0, 0)),
    out_shape=jax.ShapeDtypeStruct((B, D), jnp.float32),
    scratch_shapes=[pltpu.VMEM((B, 8, D), jnp.float32)],
)


@jax.jit
def kernel(values, prefix_sum):
    ps = jnp.pad(prefix_sum, (0, 32 - (B + 1)), mode="edge")
    out_sc = _jagged_max_sc(values, ps)
    out_tc = _tc_prefix(ps, values)
    return jnp.maximum(out_sc, out_tc)


# speedup vs baseline: 1.3780x; 1.0319x over previous
import functools

import jax
import jax.numpy as jnp
from jax import lax
from jax.experimental import pallas as pl
from jax.experimental.pallas import tpu as pltpu
from jax.experimental.pallas import tpu_sc as plsc

N = 32768
D = 512
B = 16
S_TC = 24576
SC_ROWS = N - S_TC
NC = 2
NS = 16
L = 16
CPC = D // NC
KV = CPC // L
RPT = SC_ROWS // NS
CH = 128
NCH = RPT // CH
RB = 512

_mesh = plsc.VectorSubcoreMesh(core_axis_name="c", subcore_axis_name="s")


@functools.partial(
    pl.kernel,
    mesh=_mesh,
    out_type=jax.ShapeDtypeStruct((B, D), jnp.float32),
    scratch_types=[
        pltpu.VMEM((CH, CPC), jnp.float32),
        pltpu.VMEM((CH, CPC), jnp.float32),
        pltpu.VMEM((B, CPC), jnp.float32),
        pltpu.VMEM((NS, CPC), jnp.float32),
        pltpu.VMEM((L,), jnp.int32),
        pltpu.VMEM_SHARED((B, NS, CPC), jnp.float32),
        pltpu.SemaphoreType.DMA,
        pltpu.SemaphoreType.DMA,
    ],
)
def _jagged_max_sc(values_hbm, ps_hbm, out_hbm,
                   buf0, buf1, partial, mbuf, ps_v, shared, sem0, sem1):
    cid = lax.axis_index("c")
    sid = lax.axis_index("s")
    c0 = cid * CPC
    row0 = S_TC + sid * RPT

    pltpu.sync_copy(ps_hbm.at[pl.ds(0, L)], ps_v)
    pvec0 = ps_v[pl.ds(0, L)]
    ps_s = [pvec0[i] for i in range(L)] + [N]

    neg = jnp.full((L,), -jnp.inf, jnp.float32)
    for s in range(B):
        for k in range(KV):
            partial[s, pl.ds(k * L, L)] = neg

    def start(j, buf, sem):
        pltpu.async_copy(
            values_hbm.at[pl.ds(row0 + j * CH, CH), pl.ds(c0, CPC)], buf, sem)

    def wait(buf, sem):
        pltpu.make_async_copy(
            values_hbm.at[pl.ds(row0, CH), pl.ds(c0, CPC)], buf, sem).wait()

    start(0, buf0, sem0)
    start(1, buf1, sem1)

    def process(j, buf):
        chunk_lo = row0 + j * CH
        for s in range(B):
            a = jnp.maximum(ps_s[s], chunk_lo) - chunk_lo
            b = jnp.minimum(ps_s[s + 1], chunk_lo + CH) - chunk_lo

            @pl.when(b > a)
            def _():
                acc0 = tuple(partial[s, pl.ds(k * L, L)] for k in range(KV))

                def rbody(r, acc):
                    return tuple(
                        jnp.maximum(acc[k], buf[r, pl.ds(k * L, L)])
                        for k in range(KV))

                acc = lax.fori_loop(a, b, rbody, acc0)
                for k in range(KV):
                    partial[s, pl.ds(k * L, L)] = acc[k]

    def loop_body(jj, carry):
        j = 2 * jj
        wait(buf0, sem0)
        process(j, buf0)

        @pl.when(j + 2 < NCH)
        def _():
            start(j + 2, buf0, sem0)

        wait(buf1, sem1)
        process(j + 1, buf1)

        @pl.when(j + 3 < NCH)
        def _():
            start(j + 3, buf1, sem1)

        return carry

    lax.fori_loop(0, NCH // 2, loop_body, 0)

    for s in range(B):
        pltpu.sync_copy(partial.at[s], shared.at[s, sid])
    plsc.subcore_barrier()
    pltpu.sync_copy(shared.at[sid], mbuf)
    for k in range(KV):
        acc = mbuf[0, pl.ds(k * L, L)]
        for t in range(1, NS):
            acc = jnp.maximum(acc, mbuf[t, pl.ds(k * L, L)])
        partial[0, pl.ds(k * L, L)] = acc
    pltpu.sync_copy(partial.at[0], out_hbm.at[sid, pl.ds(c0, CPC)])


def _tc_body(ps_ref, x_ref, o_ref, acc_ref):
    i = pl.program_id(0)
    nsteps = pl.num_programs(0)

    @pl.when(i == 0)
    def _():
        acc_ref[...] = jnp.full_like(acc_ref, -jnp.inf)

    base = i * RB
    for s in range(B):
        lo = ps_ref[s] if s > 0 else 0
        hi = ps_ref[s + 1] if s + 1 < B else N
        a = jnp.maximum(lo - base, 0)
        b = jnp.minimum(hi - base, RB)

        @pl.when((a == 0) & (b == RB))
        def _():
            m8 = jnp.max(x_ref[...].reshape(RB // 8, 8, D), axis=0)
            acc_ref[s] = jnp.maximum(acc_ref[s], m8)

        @pl.when((b > a) & ((a > 0) | (b < RB)))
        def _():
            iota = lax.broadcasted_iota(jnp.int32, (RB, D), 0)
            mask = (iota >= a) & (iota < b)
            masked = jnp.where(mask, x_ref[...], -jnp.inf)
            m8 = jnp.max(masked.reshape(RB // 8, 8, D), axis=0)
            acc_ref[s] = jnp.maximum(acc_ref[s], m8)

    @pl.when(i == nsteps - 1)
    def _():
        o_ref[...] = jnp.max(acc_ref[...], axis=1)


_tc_prefix = pl.pallas_call(
    _tc_body,
    grid=(S_TC // RB,),
    in_specs=[pl.BlockSpec(memory_space=pltpu.SMEM),
              pl.BlockSpec((RB, D), lambda i: (i, 0))],
    out_specs=pl.BlockSpec((B, D), lambda i: (0, 0)),
    out_shape=jax.ShapeDtypeStruct((B, D), jnp.float32),
    scratch_shapes=[pltpu.VMEM((B, 8, D), jnp.float32)],
)


@jax.jit
def kernel(values, prefix_sum):
    out_sc = _jagged_max_sc(values, prefix_sum)
    out_tc = _tc_prefix(prefix_sum, values)
    return jnp.maximum(out_sc, out_tc)


# speedup vs baseline: 1.8677x; 1.3554x over previous
import functools

import jax
import jax.numpy as jnp
from jax import lax
from jax.experimental import pallas as pl
from jax.experimental.pallas import tpu as pltpu
from jax.experimental.pallas import tpu_sc as plsc

N = 32768
D = 512
B = 16
S_TC = 24576
SC_ROWS = N - S_TC
NC = 2
NS = 16
L = 16
CPC = D // NC
KV = CPC // L
RPT = SC_ROWS // NS
CH = 128
NCH = RPT // CH
RB = 512

_mesh = plsc.VectorSubcoreMesh(core_axis_name="c", subcore_axis_name="s")


@functools.partial(
    pl.kernel,
    mesh=_mesh,
    out_type=jax.ShapeDtypeStruct((B, D), jnp.float32),
    scratch_types=[
        pltpu.VMEM((CH, CPC), jnp.float32),
        pltpu.VMEM((CH, CPC), jnp.float32),
        pltpu.VMEM((B, CPC), jnp.float32),
        pltpu.VMEM((NS, CPC), jnp.float32),
        pltpu.VMEM((L,), jnp.int32),
        pltpu.VMEM_SHARED((B, NS, CPC), jnp.float32),
        pltpu.SemaphoreType.DMA,
        pltpu.SemaphoreType.DMA,
    ],
)
def _jagged_max_sc(values_hbm, ps_hbm, out_hbm,
                   buf0, buf1, partial, mbuf, ps_v, shared, sem0, sem1):
    cid = lax.axis_index("c")
    sid = lax.axis_index("s")
    c0 = cid * CPC
    row0 = S_TC + sid * RPT

    pltpu.sync_copy(ps_hbm.at[pl.ds(0, L)], ps_v)
    pvec0 = ps_v[pl.ds(0, L)]
    ps_s = [pvec0[i] for i in range(L)] + [N]

    neg = jnp.full((L,), -jnp.inf, jnp.float32)
    for s in range(B):
        for k in range(KV):
            partial[s, pl.ds(k * L, L)] = neg

    def start(j, buf, sem):
        pltpu.async_copy(
            values_hbm.at[pl.ds(row0 + j * CH, CH), pl.ds(c0, CPC)], buf, sem)

    def wait(buf, sem):
        pltpu.make_async_copy(
            values_hbm.at[pl.ds(row0, CH), pl.ds(c0, CPC)], buf, sem).wait()

    start(0, buf0, sem0)
    start(1, buf1, sem1)

    def process(j, buf):
        chunk_lo = row0 + j * CH
        for s in range(B):
            a = jnp.maximum(ps_s[s], chunk_lo) - chunk_lo
            b = jnp.minimum(ps_s[s + 1], chunk_lo + CH) - chunk_lo

            @pl.when(b > a)
            def _():
                acc0 = tuple(partial[s, pl.ds(k * L, L)] for k in range(KV))

                def rbody(r, acc):
                    return tuple(
                        jnp.maximum(acc[k], buf[r, pl.ds(k * L, L)])
                        for k in range(KV))

                acc = lax.fori_loop(a, b, rbody, acc0)
                for k in range(KV):
                    partial[s, pl.ds(k * L, L)] = acc[k]

    def loop_body(jj, carry):
        j = 2 * jj
        wait(buf0, sem0)
        process(j, buf0)

        @pl.when(j + 2 < NCH)
        def _():
            start(j + 2, buf0, sem0)

        wait(buf1, sem1)
        process(j + 1, buf1)

        @pl.when(j + 3 < NCH)
        def _():
            start(j + 3, buf1, sem1)

        return carry

    lax.fori_loop(0, NCH // 2, loop_body, 0)

    for s in range(B):
        pltpu.sync_copy(partial.at[s], shared.at[s, sid])
    plsc.subcore_barrier()
    pltpu.sync_copy(shared.at[sid], mbuf)
    for k in range(KV):
        acc = mbuf[0, pl.ds(k * L, L)]
        for t in range(1, NS):
            acc = jnp.maximum(acc, mbuf[t, pl.ds(k * L, L)])
        partial[0, pl.ds(k * L, L)] = acc
    pltpu.sync_copy(partial.at[0], out_hbm.at[sid, pl.ds(c0, CPC)])


NBUF = 8
NBLK = S_TC // RB
NOUTER = NBLK // NBUF


def _tc_body(ps_ref, x_hbm, o_ref, bufs, acc_ref, sems):
    acc_ref[...] = jnp.full_like(acc_ref, -jnp.inf)

    def start(j, b):
        pltpu.make_async_copy(
            x_hbm.at[pl.ds(j * RB, RB), :], bufs.at[b], sems.at[b]).start()

    def wait(b):
        pltpu.make_async_copy(
            x_hbm.at[pl.ds(0, RB), :], bufs.at[b], sems.at[b]).wait()

    for b in range(NBUF):
        start(b, b)

    def process(j, buf):
        base = j * RB
        for s in range(B):
            lo = ps_ref[s] if s > 0 else 0
            hi = ps_ref[s + 1] if s + 1 < B else N
            a = jnp.maximum(lo - base, 0)
            b = jnp.minimum(hi - base, RB)

            @pl.when((a == 0) & (b == RB))
            def _():
                m8 = jnp.max(buf[...].reshape(RB // 8, 8, D), axis=0)
                acc_ref[s] = jnp.maximum(acc_ref[s], m8)

            @pl.when((b > a) & ((a > 0) | (b < RB)))
            def _():
                iota = lax.broadcasted_iota(jnp.int32, (RB, D), 0)
                mask = (iota >= a) & (iota < b)
                masked = jnp.where(mask, buf[...], -jnp.inf)
                m8 = jnp.max(masked.reshape(RB // 8, 8, D), axis=0)
                acc_ref[s] = jnp.maximum(acc_ref[s], m8)

    def outer(jj, carry):
        for b in range(NBUF):
            j = jj * NBUF + b
            wait(b)
            process(j, bufs.at[b])

            @pl.when(j + NBUF < NBLK)
            def _():
                start(j + NBUF, b)

        return carry

    lax.fori_loop(0, NOUTER, outer, 0)
    o_ref[...] = jnp.max(acc_ref[...], axis=1)


_tc_prefix = pl.pallas_call(
    _tc_body,
    in_specs=[pl.BlockSpec(memory_space=pltpu.SMEM),
              pl.BlockSpec(memory_space=pl.ANY)],
    out_shape=jax.ShapeDtypeStruct((B, D), jnp.float32),
    scratch_shapes=[pltpu.VMEM((NBUF, RB, D), jnp.float32),
                    pltpu.VMEM((B, 8, D), jnp.float32),
                    pltpu.SemaphoreType.DMA((NBUF,))],
)


@jax.jit
def kernel(values, prefix_sum):
    out_sc = _jagged_max_sc(values, prefix_sum)
    out_tc = _tc_prefix(prefix_sum, values)
    return jnp.maximum(out_sc, out_tc)


# speedup vs baseline: 1.9545x; 1.0465x over previous
import functools

import jax
import jax.numpy as jnp
from jax import lax
from jax.experimental import pallas as pl
from jax.experimental.pallas import tpu as pltpu
from jax.experimental.pallas import tpu_sc as plsc

N = 32768
D = 512
B = 16
S_TC = 24576
SC_ROWS = N - S_TC
NC = 2
NS = 16
L = 16
CPC = D // NC
KV = CPC // L
RPT = SC_ROWS // NS
CH = 128
NCH = RPT // CH
RB = 512

_mesh = plsc.VectorSubcoreMesh(core_axis_name="c", subcore_axis_name="s")


@functools.partial(
    pl.kernel,
    mesh=_mesh,
    out_type=jax.ShapeDtypeStruct((B, D), jnp.float32),
    scratch_types=[
        pltpu.VMEM((CH, CPC), jnp.float32),
        pltpu.VMEM((CH, CPC), jnp.float32),
        pltpu.VMEM((B, CPC), jnp.float32),
        pltpu.VMEM((NS, CPC), jnp.float32),
        pltpu.VMEM((L,), jnp.int32),
        pltpu.VMEM_SHARED((NS, B, CPC), jnp.float32),
        pltpu.SemaphoreType.DMA,
        pltpu.SemaphoreType.DMA,
    ],
)
def _jagged_max_sc(values_hbm, ps_hbm, out_hbm,
                   buf0, buf1, partial, mbuf, ps_v, shared, sem0, sem1):
    cid = lax.axis_index("c")
    sid = lax.axis_index("s")
    c0 = cid * CPC
    row0 = S_TC + sid * RPT

    def start(j, buf, sem):
        pltpu.async_copy(
            values_hbm.at[pl.ds(row0 + j * CH, CH), pl.ds(c0, CPC)], buf, sem)

    def wait(buf, sem):
        pltpu.make_async_copy(
            values_hbm.at[pl.ds(row0, CH), pl.ds(c0, CPC)], buf, sem).wait()

    start(0, buf0, sem0)
    start(1, buf1, sem1)

    pltpu.sync_copy(ps_hbm.at[pl.ds(0, L)], ps_v)
    pvec0 = ps_v[pl.ds(0, L)]
    ps_s = [pvec0[i] for i in range(L)] + [N]

    neg = jnp.full((L,), -jnp.inf, jnp.float32)
    for s in range(B):
        for k in range(KV):
            partial[s, pl.ds(k * L, L)] = neg

    def process(j, buf):
        chunk_lo = row0 + j * CH
        for s in range(B):
            a = jnp.maximum(ps_s[s], chunk_lo) - chunk_lo
            b = jnp.minimum(ps_s[s + 1], chunk_lo + CH) - chunk_lo

            @pl.when(b > a)
            def _():
                acc0 = tuple(partial[s, pl.ds(k * L, L)] for k in range(KV))

                def rbody(r, acc):
                    return tuple(
                        jnp.maximum(acc[k], buf[r, pl.ds(k * L, L)])
                        for k in range(KV))

                acc = lax.fori_loop(a, b, rbody, acc0)
                for k in range(KV):
                    partial[s, pl.ds(k * L, L)] = acc[k]

    def loop_body(jj, carry):
        j = 2 * jj
        wait(buf0, sem0)
        process(j, buf0)

        @pl.when(j + 2 < NCH)
        def _():
            start(j + 2, buf0, sem0)

        wait(buf1, sem1)
        process(j + 1, buf1)

        @pl.when(j + 3 < NCH)
        def _():
            start(j + 3, buf1, sem1)

        return carry

    lax.fori_loop(0, NCH // 2, loop_body, 0)

    pltpu.sync_copy(partial, shared.at[sid])
    plsc.subcore_barrier()
    pltpu.sync_copy(shared.at[:, sid], mbuf)
    for k in range(KV):
        acc = mbuf[0, pl.ds(k * L, L)]
        for t in range(1, NS):
            acc = jnp.maximum(acc, mbuf[t, pl.ds(k * L, L)])
        partial[0, pl.ds(k * L, L)] = acc
    pltpu.sync_copy(partial.at[0], out_hbm.at[sid, pl.ds(c0, CPC)])


NBUF = 8
NBLK = S_TC // RB
NOUTER = NBLK // NBUF


def _tc_body(ps_ref, x_hbm, o_ref, bufs, acc_ref, sems):
    acc_ref[...] = jnp.full_like(acc_ref, -jnp.inf)

    def start(j, b):
        pltpu.make_async_copy(
            x_hbm.at[pl.ds(j * RB, RB), :], bufs.at[b], sems.at[b]).start()

    def wait(b):
        pltpu.make_async_copy(
            x_hbm.at[pl.ds(0, RB), :], bufs.at[b], sems.at[b]).wait()

    for b in range(NBUF):
        start(b, b)

    def process(j, buf):
        base = j * RB
        for s in range(B):
            lo = ps_ref[s] if s > 0 else 0
            hi = ps_ref[s + 1] if s + 1 < B else N
            a = jnp.maximum(lo - base, 0)
            b = jnp.minimum(hi - base, RB)

            @pl.when((a == 0) & (b == RB))
            def _():
                m8 = jnp.max(buf[...].reshape(RB // 8, 8, D), axis=0)
                acc_ref[s] = jnp.maximum(acc_ref[s], m8)

            @pl.when((b > a) & ((a > 0) | (b < RB)))
            def _():
                iota = lax.broadcasted_iota(jnp.int32, (RB, D), 0)
                mask = (iota >= a) & (iota < b)
                masked = jnp.where(mask, buf[...], -jnp.inf)
                m8 = jnp.max(masked.reshape(RB // 8, 8, D), axis=0)
                acc_ref[s] = jnp.maximum(acc_ref[s], m8)

    def outer(jj, carry):
        for b in range(NBUF):
            j = jj * NBUF + b
            wait(b)
            process(j, bufs.at[b])

            @pl.when(j + NBUF < NBLK)
            def _():
                start(j + NBUF, b)

        return carry

    lax.fori_loop(0, NOUTER, outer, 0)
    o_ref[...] = jnp.max(acc_ref[...], axis=1)


_tc_prefix = pl.pallas_call(
    _tc_body,
    in_specs=[pl.BlockSpec(memory_space=pltpu.SMEM),
              pl.BlockSpec(memory_space=pl.ANY)],
    out_shape=jax.ShapeDtypeStruct((B, D), jnp.float32),
    scratch_shapes=[pltpu.VMEM((NBUF, RB, D), jnp.float32),
                    pltpu.VMEM((B, 8, D), jnp.float32),
                    pltpu.SemaphoreType.DMA((NBUF,))],
)


@jax.jit
def kernel(values, prefix_sum):
    out_sc = _jagged_max_sc(values, prefix_sum)
    out_tc = _tc_prefix(prefix_sum, values)
    return jnp.maximum(out_sc, out_tc)
